# baseline (device time: 68127 ns/iter reference)
import jax
import jax.numpy as jnp
from jax import lax
from jax.experimental import pallas as pl
from jax.experimental.pallas import tpu as pltpu

N_DEV = 4
SQ = 1024
SKV = 1024
HQ_LOCAL = 8
DH = 128
D_MODEL = 1024
D_LOCAL = HQ_LOCAL * DH
BLK = 64
L = 128
NSUB = SQ // L
SCALE = 0.08838834764831843


def _body(x_ref, wq_ref, k_ref, v_ref, wo_ref, out_ref,
          partbf_ref, rs_recv_ref, red_ref, ag_recv_ref,
          rs_send_sems, rs_recv_sems, ag_send_sems, ag_recv_sems):
    my = lax.axis_index("i")

    barrier = pltpu.get_barrier_semaphore()
    for o in (1, 2, 3):
        pl.semaphore_signal(barrier, inc=1, device_id=((my + o) % N_DEV,),
                            device_id_type=pl.DeviceIdType.MESH)
    pl.semaphore_wait(barrier, 3)

    q = jnp.dot(x_ref[...], wq_ref[...], preferred_element_type=jnp.float32)
    q = (q * SCALE).astype(jnp.bfloat16)

    rowb = lax.broadcasted_iota(jnp.int32, (L, L), 0) // BLK
    colb = lax.broadcasted_iota(jnp.int32, (L, L), 1) // BLK
    bias_diag = jnp.where(colb <= rowb, 0.0, -1e9).astype(jnp.float32)

    accs = [None] * NSUB

    def compute_subchunk(g):
        K = (g + 1) * L
        fb = bias_diag if g == 0 else jnp.concatenate(
            [jnp.zeros((L, K - L), jnp.float32), bias_diag], axis=1)
        qg = q[g * L:(g + 1) * L, :]
        acc = jnp.zeros((L, D_MODEL), jnp.float32)
        for h in range(HQ_LOCAL):
            qh = qg[:, h * DH:(h + 1) * DH]
            kh = k_ref[:K, h * DH:(h + 1) * DH]
            s = lax.dot_general(qh, kh, (((1,), (1,)), ((), ())),
                                preferred_element_type=jnp.float32)
            e = jnp.exp(s + fb)
            denom = jnp.sum(e, axis=1, keepdims=True)
            ctx = jnp.dot(e.astype(jnp.bfloat16), v_ref[:K, h * DH:(h + 1) * DH],
                          preferred_element_type=jnp.float32)
            ctx = (ctx / denom).astype(jnp.bfloat16)
            acc = acc + jnp.dot(ctx, wo_ref[h * DH:(h + 1) * DH, :],
                                preferred_element_type=jnp.float32)
        accs[g] = acc
        partbf_ref[g] = acc.astype(jnp.bfloat16)

        owner = g // 2

        @pl.when(owner != my)
        def _send():
            slot_r = ((my - owner + N_DEV) % N_DEV - 1) * 2 + (g % 2)
            pltpu.make_async_remote_copy(
                src_ref=partbf_ref.at[g],
                dst_ref=rs_recv_ref.at[slot_r],
                send_sem=rs_send_sems.at[g],
                recv_sem=rs_recv_sems.at[slot_r],
                device_id=(owner,),
                device_id_type=pl.DeviceIdType.MESH,
            ).start()

    def reduce_and_broadcast(g):
        owner = g // 2

        @pl.when(my == owner)
        def _():
            red = accs[g]
            for r in (1, 2, 3):
                slot = (r - 1) * 2 + (g % 2)
                pltpu.make_async_remote_copy(
                    src_ref=rs_recv_ref.at[slot],
                    dst_ref=rs_recv_ref.at[slot],
                    send_sem=rs_send_sems.at[0],
                    recv_sem=rs_recv_sems.at[slot],
                    device_id=(owner,),
                    device_id_type=pl.DeviceIdType.MESH,
                ).wait_recv()
                red = red + rs_recv_ref[slot].astype(jnp.float32)
            out_ref[0, g * L:(g + 1) * L, :] = red
            red_ref[g % 2] = red.astype(jnp.bfloat16)
            for o in (1, 2, 3):
                peer = (owner + o) % N_DEV
                slot_r = (3 - o) * 2 + (g % 2)
                pltpu.make_async_remote_copy(
                    src_ref=red_ref.at[g % 2],
                    dst_ref=ag_recv_ref.at[slot_r],
                    send_sem=ag_send_sems.at[(o - 1) * 2 + (g % 2)],
                    recv_sem=ag_recv_sems.at[slot_r],
                    device_id=(peer,),
                    device_id_type=pl.DeviceIdType.MESH,
                ).start()

    for p in range(NSUB + 2):
        if p < NSUB:
            compute_subchunk(p)
        if p >= 2:
            reduce_and_broadcast(p - 2)

    for r in (1, 2, 3):
        for sub in (0, 1):
            slot = (r - 1) * 2 + sub
            pltpu.make_async_remote_copy(
                src_ref=ag_recv_ref.at[slot],
                dst_ref=ag_recv_ref.at[slot],
                send_sem=ag_send_sems.at[0],
                recv_sem=ag_recv_sems.at[slot],
                device_id=(my,),
                device_id_type=pl.DeviceIdType.MESH,
            ).wait_recv()
            row0 = ((my + r) % N_DEV) * (2 * L) + sub * L
            out_ref[0, pl.ds(row0, L), :] = ag_recv_ref[slot].astype(jnp.float32)

    for g in range(NSUB):
        @pl.when(g // 2 != my)
        def _(g=g):
            pltpu.make_async_remote_copy(
                src_ref=partbf_ref.at[g],
                dst_ref=rs_recv_ref.at[0],
                send_sem=rs_send_sems.at[g],
                recv_sem=rs_recv_sems.at[0],
                device_id=(my,),
                device_id_type=pl.DeviceIdType.MESH,
            ).wait_send()
    for i in range(6):
        pltpu.make_async_remote_copy(
            src_ref=red_ref.at[i % 2],
            dst_ref=ag_recv_ref.at[0],
            send_sem=ag_send_sems.at[i],
            recv_sem=ag_recv_sems.at[0],
            device_id=(my,),
            device_id_type=pl.DeviceIdType.MESH,
        ).wait_send()


def kernel(x, Wq, K_ext, V_ext, Wo):
    idx = lax.axis_index("i")
    x2 = x[0].astype(jnp.bfloat16)
    wq_l = lax.dynamic_slice(Wq, (0, idx * D_LOCAL),
                             (D_MODEL, D_LOCAL)).astype(jnp.bfloat16)
    wo_l = lax.dynamic_slice(Wo, (idx * D_LOCAL, 0),
                             (D_LOCAL, D_MODEL)).astype(jnp.bfloat16)
    k2 = K_ext[0].reshape(SKV, D_LOCAL).astype(jnp.bfloat16)
    v2 = V_ext[0].reshape(SKV, D_LOCAL).astype(jnp.bfloat16)

    return pl.pallas_call(
        _body,
        out_shape=jax.ShapeDtypeStruct((1, SQ, D_MODEL), jnp.float32),
        in_specs=[pl.BlockSpec(memory_space=pltpu.VMEM)] * 5,
        out_specs=pl.BlockSpec(memory_space=pltpu.VMEM),
        scratch_shapes=[
            pltpu.VMEM((NSUB, L, D_MODEL), jnp.bfloat16),
            pltpu.VMEM((6, L, D_MODEL), jnp.bfloat16),
            pltpu.VMEM((2, L, D_MODEL), jnp.bfloat16),
            pltpu.VMEM((6, L, D_MODEL), jnp.bfloat16),
            pltpu.SemaphoreType.DMA((NSUB,)),
            pltpu.SemaphoreType.DMA((6,)),
            pltpu.SemaphoreType.DMA((6,)),
            pltpu.SemaphoreType.DMA((6,)),
        ],
        compiler_params=pltpu.CompilerParams(collective_id=0),
    )(x2, wq_l, k2, v2, wo_l)


# device time: 45660 ns/iter; 1.4920x vs baseline; 1.4920x over previous
import jax
import jax.numpy as jnp
from jax import lax
from jax.experimental import pallas as pl
from jax.experimental.pallas import tpu as pltpu

N_DEV = 4
SQ = 1024
SKV = 1024
HQ_LOCAL = 8
DH = 128
D_MODEL = 1024
D_LOCAL = HQ_LOCAL * DH
BLK = 64
L = 256
RB = L // N_DEV
SCALE = 0.08838834764831843


def _body(x_ref, wq_ref, k_ref, v_ref, wo_ref, out_ref,
          partbf_ref, rs_recv_ref, red_ref, ag_recv_ref,
          rs_send_sems, rs_recv_sems, ag_send_sems, ag_recv_sems):
    my = lax.axis_index("i")

    barrier = pltpu.get_barrier_semaphore()
    for o in (1, 2, 3):
        pl.semaphore_signal(barrier, inc=1, device_id=((my + o) % N_DEV,),
                            device_id_type=pl.DeviceIdType.MESH)
    pl.semaphore_wait(barrier, 3)

    q = jnp.dot(x_ref[...], wq_ref[...], preferred_element_type=jnp.float32)
    q = (q * SCALE).astype(jnp.bfloat16)

    rowb = lax.broadcasted_iota(jnp.int32, (L, L), 0) // BLK
    colb = lax.broadcasted_iota(jnp.int32, (L, L), 1) // BLK
    bias_diag = jnp.where(colb <= rowb, 0.0, -1e9).astype(jnp.float32)

    def compute_chunk(c):
        K = (c + 1) * L
        fb = bias_diag if c == 0 else jnp.concatenate(
            [jnp.zeros((L, K - L), jnp.float32), bias_diag], axis=1)
        qc = q[c * L:(c + 1) * L, :]
        acc = jnp.zeros((L, D_MODEL), jnp.float32)
        for h in range(HQ_LOCAL):
            qh = qc[:, h * DH:(h + 1) * DH]
            kh = k_ref[:K, h * DH:(h + 1) * DH]
            s = lax.dot_general(qh, kh, (((1,), (1,)), ((), ())),
                                preferred_element_type=jnp.float32)
            e = jnp.exp(s + fb)
            denom = jnp.sum(e, axis=1, keepdims=True)
            ctx = jnp.dot(e.astype(jnp.bfloat16), v_ref[:K, h * DH:(h + 1) * DH],
                          preferred_element_type=jnp.float32)
            ctx = (ctx / denom).astype(jnp.bfloat16)
            acc = acc + jnp.dot(ctx, wo_ref[h * DH:(h + 1) * DH, :],
                                preferred_element_type=jnp.float32)
        partbf_ref[c] = acc.astype(jnp.bfloat16)

        for o in (1, 2, 3):
            peer = (my + o) % N_DEV
            pltpu.make_async_remote_copy(
                src_ref=partbf_ref.at[c, pl.ds(peer * RB, RB)],
                dst_ref=rs_recv_ref.at[c, 3 - o],
                send_sem=rs_send_sems.at[c, o - 1],
                recv_sem=rs_recv_sems.at[c, 3 - o],
                device_id=(peer,),
                device_id_type=pl.DeviceIdType.MESH,
            ).start()

    def reduce_and_broadcast(c):
        red = partbf_ref[c, pl.ds(my * RB, RB), :].astype(jnp.float32)
        for slot in range(3):
            pltpu.make_async_remote_copy(
                src_ref=rs_recv_ref.at[c, slot],
                dst_ref=rs_recv_ref.at[c, slot],
                send_sem=rs_send_sems.at[c, 0],
                recv_sem=rs_recv_sems.at[c, slot],
                device_id=(my,),
                device_id_type=pl.DeviceIdType.MESH,
            ).wait_recv()
            red = red + rs_recv_ref[c, slot].astype(jnp.float32)
        out_ref[0, pl.ds(c * L + my * RB, RB), :] = red
        red_ref[c] = red.astype(jnp.bfloat16)
        for o in (1, 2, 3):
            pltpu.make_async_remote_copy(
                src_ref=red_ref.at[c],
                dst_ref=ag_recv_ref.at[c, 3 - o],
                send_sem=ag_send_sems.at[c, o - 1],
                recv_sem=ag_recv_sems.at[c, 3 - o],
                device_id=((my + o) % N_DEV,),
                device_id_type=pl.DeviceIdType.MESH,
            ).start()

    for p in range(N_DEV):
        compute_chunk(p)
        if p >= 1:
            reduce_and_broadcast(p - 1)
    reduce_and_broadcast(N_DEV - 1)

    for c in range(N_DEV):
        for s in range(3):
            pltpu.make_async_remote_copy(
                src_ref=ag_recv_ref.at[c, s],
                dst_ref=ag_recv_ref.at[c, s],
                send_sem=ag_send_sems.at[c, 0],
                recv_sem=ag_recv_sems.at[c, s],
                device_id=(my,),
                device_id_type=pl.DeviceIdType.MESH,
            ).wait_recv()
            row0 = c * L + ((my + s + 1) % N_DEV) * RB
            out_ref[0, pl.ds(row0, RB), :] = ag_recv_ref[c, s].astype(jnp.float32)

    for c in range(N_DEV):
        for i in range(3):
            pltpu.make_async_remote_copy(
                src_ref=rs_recv_ref.at[c, 0],
                dst_ref=rs_recv_ref.at[c, 0],
                send_sem=rs_send_sems.at[c, i],
                recv_sem=rs_recv_sems.at[c, 0],
                device_id=(my,),
                device_id_type=pl.DeviceIdType.MESH,
            ).wait_send()
            pltpu.make_async_remote_copy(
                src_ref=red_ref.at[c],
                dst_ref=ag_recv_ref.at[c, 0],
                send_sem=ag_send_sems.at[c, i],
                recv_sem=ag_recv_sems.at[c, 0],
                device_id=(my,),
                device_id_type=pl.DeviceIdType.MESH,
            ).wait_send()


def kernel(x, Wq, K_ext, V_ext, Wo):
    idx = lax.axis_index("i")
    x2 = x[0].astype(jnp.bfloat16)
    wq_l = lax.dynamic_slice(Wq, (0, idx * D_LOCAL),
                             (D_MODEL, D_LOCAL)).astype(jnp.bfloat16)
    wo_l = lax.dynamic_slice(Wo, (idx * D_LOCAL, 0),
                             (D_LOCAL, D_MODEL)).astype(jnp.bfloat16)
    k2 = K_ext[0].reshape(SKV, D_LOCAL).astype(jnp.bfloat16)
    v2 = V_ext[0].reshape(SKV, D_LOCAL).astype(jnp.bfloat16)

    return pl.pallas_call(
        _body,
        out_shape=jax.ShapeDtypeStruct((1, SQ, D_MODEL), jnp.float32),
        in_specs=[pl.BlockSpec(memory_space=pltpu.VMEM)] * 5,
        out_specs=pl.BlockSpec(memory_space=pltpu.VMEM),
        scratch_shapes=[
            pltpu.VMEM((N_DEV, L, D_MODEL), jnp.bfloat16),
            pltpu.VMEM((N_DEV, 3, RB, D_MODEL), jnp.bfloat16),
            pltpu.VMEM((N_DEV, RB, D_MODEL), jnp.bfloat16),
            pltpu.VMEM((N_DEV, 3, RB, D_MODEL), jnp.bfloat16),
            pltpu.SemaphoreType.DMA((N_DEV, 3)),
            pltpu.SemaphoreType.DMA((N_DEV, 3)),
            pltpu.SemaphoreType.DMA((N_DEV, 3)),
            pltpu.SemaphoreType.DMA((N_DEV, 3)),
        ],
        compiler_params=pltpu.CompilerParams(collective_id=0),
    )(x2, wq_l, k2, v2, wo_l)
